# Initial kernel scaffold; baseline (speedup 1.0000x reference)
import jax
import jax.numpy as jnp
from jax.experimental import pallas as pl


def kernel(scene_x, graph_x, W, b):
    raise NotImplementedError("write your pallas kernel here")


# speedup vs baseline: 2.1656x; 2.1656x over previous
import functools

import jax
import jax.numpy as jnp
from jax import lax
from jax.experimental import pallas as pl
from jax.experimental.pallas import tpu as pltpu
from jax.experimental.pallas import tpu_sc as plsc

_TOPS = 10
_SAMP = 64
_TEMP = 0.07
_NEG = float("-inf")


def _pick_bk(k_total):
    for bk in range(min(k_total, 2048), 7, -1):
        if k_total % bk == 0 and bk % 8 == 0:
            return bk
    return k_total


def _phase_a_body(scene_ref, graph_ref, md_ref, y_ref, smean_ref, carry_ref):
    nq = scene_ref.shape[0]
    bk = graph_ref.shape[0]
    nblk = pl.num_programs(0)
    k = pl.program_id(0)

    @pl.when(k == 0)
    def _init():
        carry_ref[...] = jnp.full((nq, 16), _NEG, jnp.float32)
        smean_ref[...] = jnp.mean(scene_ref[...], axis=0, keepdims=True)

    sim = lax.dot_general(
        scene_ref[...], graph_ref[...],
        dimension_numbers=(((1,), (1,)), ((), ())),
        preferred_element_type=jnp.float32)

    li = lax.broadcasted_iota(jnp.int32, (nq, bk), 1)
    sl = lax.broadcasted_iota(jnp.int32, (nq, 16), 1)
    carry0 = carry_ref[...]

    def cond(state):
        _x, _c, anyq, i = state
        return jnp.logical_and(anyq, i < _TOPS)

    def body(state):
        x, c, _aq, i = state
        m = jnp.max(x, axis=1, keepdims=True)
        idx = jnp.min(jnp.where(x == m, li, bk), axis=1, keepdims=True)
        x = jnp.where(li == idx, _NEG, x)
        pos = jnp.sum((c > m).astype(jnp.int32), axis=1, keepdims=True)
        sv = jnp.concatenate(
            [jnp.full((nq, 1), _NEG, jnp.float32), c[:, :15]], axis=1)
        c = jnp.where(sl < pos, c, jnp.where(sl == pos, m, sv))
        t = c[:, _TOPS - 1:_TOPS]
        anyq = jnp.any(jnp.max(x, axis=1, keepdims=True) > t)
        return x, c, anyq, i + 1

    t0 = carry0[:, _TOPS - 1:_TOPS]
    anyq0 = jnp.any(jnp.max(sim, axis=1, keepdims=True) > t0)
    _, carry1, _, _ = lax.while_loop(
        cond, body, (sim, carry0, anyq0, jnp.int32(0)))
    carry_ref[...] = carry1

    @pl.when(k == nblk - 1)
    def _final():
        c = carry_ref[...]
        md_ref[...] = jnp.mean(c[:, :_TOPS], axis=1, keepdims=True)
        best = c[:, 0:1]
        gm = jnp.max(best)
        ri = lax.broadcasted_iota(jnp.int32, (nq, 1), 0)
        rid = jnp.min(jnp.where(best == gm, ri, nq))
        oh = (lax.broadcasted_iota(jnp.int32, (1, nq), 1) == rid
              ).astype(jnp.float32)
        y_ref[...] = lax.dot_general(
            oh, scene_ref[...],
            dimension_numbers=(((1,), (0,)), ((), ())),
            preferred_element_type=jnp.float32)


def _phase_a(scene_x, graph_x, bk):
    nq, d = scene_x.shape
    nblk = graph_x.shape[0] // bk
    return pl.pallas_call(
        _phase_a_body,
        grid=(nblk,),
        in_specs=[
            pl.BlockSpec((nq, d), lambda k: (0, 0)),
            pl.BlockSpec((bk, d), lambda k: (k, 0)),
        ],
        out_specs=[
            pl.BlockSpec((nq, 1), lambda k: (0, 0)),
            pl.BlockSpec((1, d), lambda k: (0, 0)),
            pl.BlockSpec((1, d), lambda k: (0, 0)),
        ],
        out_shape=[
            jax.ShapeDtypeStruct((nq, 1), jnp.float32),
            jax.ShapeDtypeStruct((1, d), jnp.float32),
            jax.ShapeDtypeStruct((1, d), jnp.float32),
        ],
        scratch_shapes=[pltpu.VMEM((nq, 16), jnp.float32)],
    )(scene_x, graph_x)


def _phase_b_body(graph_ref, y_ref, cidx_ref, vals_ref, idxs_ref):
    bk, d = graph_ref.shape
    nblk = pl.num_programs(0)
    k = pl.program_id(0)

    @pl.when(k == 0)
    def _init():
        vals_ref[...] = jnp.full((1, _SAMP), _NEG, jnp.float32)
        idxs_ref[...] = jnp.zeros((1, _SAMP), jnp.int32)

    g = graph_ref[...]
    y = y_ref[...]
    y8 = jnp.broadcast_to(y, (8, d))
    dots = lax.dot_general(
        y8, g, dimension_numbers=(((1,), (1,)), ((), ())),
        preferred_element_type=jnp.float32)
    sqn = lax.dot_general(
        jnp.ones((8, d), jnp.float32), g * g,
        dimension_numbers=(((1,), (1,)), ((), ())),
        preferred_element_type=jnp.float32)
    ynorm = jnp.sqrt(jnp.sum(y * y))
    cs = dots / (jnp.sqrt(sqn) * ynorm + 1e-8)
    x0 = cs[0:1, :]

    li = lax.broadcasted_iota(jnp.int32, (1, bk), 1)
    sl = lax.broadcasted_iota(jnp.int32, (1, _SAMP), 1)
    vals0 = vals_ref[...]
    idxs0 = idxs_ref[...]

    def cond(state):
        _x, _v, _ii, anyq, i = state
        return jnp.logical_and(anyq, i < _SAMP)

    def body(state):
        x, v, ii, _aq, i = state
        m = jnp.max(x)
        pidx = jnp.min(jnp.where(x == m, li, bk))
        gidx = k * bk + pidx
        x = jnp.where(li == pidx, _NEG, x)
        pos = jnp.sum((v > m).astype(jnp.int32))
        svv = jnp.concatenate(
            [jnp.full((1, 1), _NEG, jnp.float32), v[:, :_SAMP - 1]], axis=1)
        sii = jnp.concatenate(
            [jnp.zeros((1, 1), jnp.int32), ii[:, :_SAMP - 1]], axis=1)
        v = jnp.where(sl < pos, v, jnp.where(sl == pos, m, svv))
        ii = jnp.where(sl < pos, ii, jnp.where(sl == pos, gidx, sii))
        t = jnp.sum(v[:, _SAMP - 1:_SAMP])
        anyq = jnp.max(x) > t
        return x, v, ii, anyq, i + 1

    t0 = jnp.sum(vals0[:, _SAMP - 1:_SAMP])
    anyq0 = jnp.max(x0) > t0
    _, vals1, idxs1, _, _ = lax.while_loop(
        cond, body, (x0, vals0, idxs0, anyq0, jnp.int32(0)))
    vals_ref[...] = vals1
    idxs_ref[...] = idxs1

    @pl.when(k == nblk - 1)
    def _final():
        cidx_ref[...] = idxs_ref[...]


def _phase_b(graph_x, y, bk):
    kk, d = graph_x.shape
    nblk = kk // bk
    return pl.pallas_call(
        _phase_b_body,
        grid=(nblk,),
        in_specs=[
            pl.BlockSpec((bk, d), lambda k: (k, 0)),
            pl.BlockSpec((1, d), lambda k: (0, 0)),
        ],
        out_specs=pl.BlockSpec((1, _SAMP), lambda k: (0, 0)),
        out_shape=jax.ShapeDtypeStruct((1, _SAMP), jnp.int32),
        scratch_shapes=[
            pltpu.VMEM((1, _SAMP), jnp.float32),
            pltpu.VMEM((1, _SAMP), jnp.int32),
        ],
    )(graph_x, y)


def _gather_rows(graph_x, cidx):
    d = graph_x.shape[1]
    info = plsc.get_sparse_core_info()
    nc = info.num_cores
    mesh = plsc.VectorSubcoreMesh(core_axis_name="c", subcore_axis_name="s")

    @functools.partial(
        pl.kernel, mesh=mesh,
        out_type=jax.ShapeDtypeStruct((_SAMP, d), jnp.float32),
        scratch_types=[
            pltpu.VMEM((8,), jnp.int32),
            pltpu.VMEM((8, d), jnp.float32),
            pltpu.SemaphoreType.DMA,
        ],
    )
    def gk(table_hbm, idx_hbm, out_hbm, idx_v, rows_v, sem):
        wid = lax.axis_index("s") * nc + lax.axis_index("c")

        @pl.when(wid < _SAMP // 8)
        def _():
            base = wid * 8
            pltpu.sync_copy(idx_hbm.at[pl.ds(base, 8)], idx_v)
            pltpu.async_copy(table_hbm.at[idx_v], rows_v, sem).wait()
            pltpu.sync_copy(rows_v, out_hbm.at[pl.ds(base, 8)])

    return gk(graph_x, cidx)


def _phase_c_body(sub_ref, smean_ref, w_ref, b_ref, tmv_ref, tsub_ref,
                  semb_ref):
    w = w_ref[...]
    b = b_ref[...]
    semb = jnp.maximum(
        lax.dot_general(smean_ref[...], w,
                        dimension_numbers=(((1,), (0,)), ((), ())),
                        preferred_element_type=jnp.float32) + b, 0.0)
    semb_ref[...] = semb
    sp = jnp.maximum(
        lax.dot_general(sub_ref[...], w,
                        dimension_numbers=(((1,), (0,)), ((), ())),
                        preferred_element_type=jnp.float32) + b, 0.0)
    logits = lax.dot_general(
        sp, semb, dimension_numbers=(((1,), (1,)), ((), ())),
        preferred_element_type=jnp.float32) / _TEMP
    match = jax.nn.sigmoid(logits)

    ri = lax.broadcasted_iota(jnp.int32, (_SAMP, 1), 0)
    oh_r = lax.broadcasted_iota(jnp.int32, (_TOPS, _SAMP), 0)
    oh_c = lax.broadcasted_iota(jnp.int32, (_TOPS, _SAMP), 1)
    tm_r = lax.broadcasted_iota(jnp.int32, (_TOPS, 1), 0)
    x = match
    oh = jnp.zeros((_TOPS, _SAMP), jnp.float32)
    tm = jnp.zeros((_TOPS, 1), jnp.float32)
    for t in range(_TOPS):
        m = jnp.max(x)
        idx = jnp.min(jnp.where(x == m, ri, _SAMP))
        x = jnp.where(ri == idx, _NEG, x)
        oh = oh + jnp.where(
            jnp.logical_and(oh_r == t, oh_c == idx), 1.0, 0.0)
        tm = tm + jnp.where(tm_r == t, m, 0.0)
    tmv_ref[...] = tm
    tsub_ref[...] = lax.dot_general(
        oh, sp, dimension_numbers=(((1,), (0,)), ((), ())),
        preferred_element_type=jnp.float32)


def _phase_c(sub_embs, smean, w, b2):
    d, lout = w.shape
    return pl.pallas_call(
        _phase_c_body,
        in_specs=[
            pl.BlockSpec((_SAMP, d), lambda: (0, 0)),
            pl.BlockSpec((1, d), lambda: (0, 0)),
            pl.BlockSpec((d, lout), lambda: (0, 0)),
            pl.BlockSpec((1, lout), lambda: (0, 0)),
        ],
        out_specs=[
            pl.BlockSpec((_TOPS, 1), lambda: (0, 0)),
            pl.BlockSpec((_TOPS, lout), lambda: (0, 0)),
            pl.BlockSpec((1, lout), lambda: (0, 0)),
        ],
        out_shape=[
            jax.ShapeDtypeStruct((_TOPS, 1), jnp.float32),
            jax.ShapeDtypeStruct((_TOPS, lout), jnp.float32),
            jax.ShapeDtypeStruct((1, lout), jnp.float32),
        ],
    )(sub_embs, smean, w, b2)


def kernel(scene_x, graph_x, W, b):
    kk = graph_x.shape[0]
    bk = _pick_bk(kk)
    md, y, smean = _phase_a(scene_x, graph_x, bk)
    cidx = _phase_b(graph_x, y, bk)
    sub_embs = _gather_rows(graph_x, cidx.reshape(_SAMP))
    tmv, tsub, semb = _phase_c(sub_embs, smean, W, b.reshape(1, -1))
    return (tmv.reshape(_TOPS), tsub, semb.reshape(-1), md.reshape(-1))
